# baseline (speedup 1.0000x reference)
import jax
import jax.numpy as jnp
from jax import lax
from jax.experimental import pallas as pl
from jax.experimental.pallas import tpu as pltpu
from jax.experimental.pallas import tpu_sc as plsc

_ALPHA = 6.26
_D = 32768
_B = 128
_L = 16
_GROUPS = 128
_VPG = 16
_NSUB = 32
_ROWS_PER_W = _B // _NSUB
_GV = _GROUPS // _L


def _iota16():
    return lax.broadcasted_iota(jnp.int32, (_L,), 0)


def _splat_i32(x):
    return jnp.broadcast_to(x, (_L,)).astype(jnp.int32)


def _splat_f32(x):
    return jnp.broadcast_to(x, (_L,)).astype(jnp.float32)


def _store1(ref, pos, val, i16):
    plsc.store_scatter(ref, [_splat_i32(pos)], val, mask=i16 == 0)


def _sc_body(x_hbm, o_hbm, row_a, row_b, out_v, maxbuf, minbuf,
             pos_a, pos_b, neg_a, neg_b, nval_a, nval_b,
             sem_a, sem_b, osem):
    i16 = _iota16()
    wid = lax.axis_index("c") * 16 + lax.axis_index("s")
    rows = (row_a, row_b)
    plists = (pos_a, pos_b)
    nlists = (neg_a, neg_b)
    vlists = (nval_a, nval_b)
    sems = (sem_a, sem_b)

    in_copies = [pltpu.async_copy(x_hbm.at[wid * _ROWS_PER_W], row_a, sem_a)]

    @plsc.parallel_loop(0, _GROUPS, 1, unroll=2)
    def _zero_block(i):
        for k in range(_VPG):
            out_v[pl.ds(i * (_VPG * _L) + k * _L, _L)] = \
                jnp.zeros((_L,), jnp.float32)

    out_copy = [None]

    for j in range(_ROWS_PER_W):
        row = wid * _ROWS_PER_W + j
        row_v = rows[j % 2]
        posidx = plists[j % 2]
        negidx = nlists[j % 2]
        negval = vlists[j % 2]
        in_copies.pop().wait()
        if j + 1 < _ROWS_PER_W:
            in_copies.append(pltpu.async_copy(
                x_hbm.at[row + 1], rows[(j + 1) % 2], sems[(j + 1) % 2]))

        zv = jnp.zeros((_L,), jnp.float32)

        def p1_group(g, carry):
            s_all, s_abs, m_all, mn_all = carry
            base = g * (_VPG * _L)
            v = row_v[pl.ds(base, _L)]
            gmx = v
            gmn = v
            sa = v
            sb = jnp.abs(v)
            for k in range(1, _VPG):
                v = row_v[pl.ds(base + k * _L, _L)]
                gmx = jnp.maximum(gmx, v)
                gmn = jnp.minimum(gmn, v)
                sa = sa + v
                sb = sb + jnp.abs(v)
            plsc.store_scatter(maxbuf, [i16 * _GROUPS + g], gmx)
            plsc.store_scatter(minbuf, [i16 * _GROUPS + g], gmn)
            return (s_all + sa, s_abs + sb,
                    jnp.maximum(m_all, gmx), jnp.minimum(mn_all, gmn))

        s_all, s_abs, m_all, mn_all = plsc.parallel_loop(
            0, _GROUPS, 1, unroll=2,
            carry=(zv, zv, jnp.full((_L,), -jnp.inf, jnp.float32),
                   jnp.full((_L,), jnp.inf, jnp.float32)))(p1_group)
        sum_all = jnp.sum(s_all)
        sum_abs = jnp.sum(s_abs)
        s_pos = 0.5 * (sum_abs + sum_all)
        s_negsum = 0.5 * (sum_all - sum_abs)

        def side(buf, m_vec, idx_list, val_list, is_max, i, acc):
            red = jnp.max if is_max else jnp.min
            m = red(m_vec)
            l_v = plsc.all_reduce_ffs(m_vec == m)
            found = jnp.full((_L,), _GROUPS, jnp.int32)
            gvs = []
            for t in range(_GV):
                gvec = plsc.load_gather(
                    buf, [l_v * _GROUPS + (t * _L) + i16])
                gvs.append(gvec)
                found = jnp.minimum(
                    found, jnp.where(gvec == m, i16 + t * _L, _GROUPS))
            g_v = _splat_i32(jnp.min(found))
            base_v = g_v * (_VPG * _L) + l_v
            didx = base_v + i16 * _L
            v = plsc.load_gather(row_v, [didx])
            kl_v = plsc.all_reduce_ffs(v == m)
            elem_v = base_v + kl_v * _L
            v2 = jnp.where(i16 == kl_v, 0.0, v)
            plsc.store_scatter(row_v, [didx], v2)
            newm = _splat_f32(red(v2))
            _store1(buf, l_v * _GROUPS + g_v, newm, i16)
            _store1(idx_list, i, elem_v, i16)
            if val_list is not None:
                _store1(val_list, i, _splat_f32(m), i16)
            lacc = None
            for t in range(_GV):
                gm = jnp.where(t * _L + i16 == g_v, newm, gvs[t])
                lacc = gm if lacc is None else (
                    jnp.maximum(lacc, gm) if is_max
                    else jnp.minimum(lacc, gm))
            ml = red(lacc)
            m_vec = jnp.where(i16 == l_v, ml, m_vec)
            acc = acc + (elem_v.astype(jnp.float32) if is_max
                         else _splat_f32(m))
            return m_vec, acc

        def step(i, carry):
            mx_vec, accp, mn_vec, accn = carry
            mx_vec, accp = side(maxbuf, mx_vec, posidx, None, True, i, accp)
            mn_vec, accn = side(minbuf, mn_vec, negidx, negval, False, i,
                                accn)
            return (mx_vec, accp, mn_vec, accn)

        _, accp, _, accn = lax.fori_loop(
            0, 32, step, (m_all, zv, mn_all, zv))
        sum_idx = jnp.max(accp)
        sum_m = jnp.min(accn)

        pos_tmp = _ALPHA * (s_pos - sum_idx)
        neg_tmp = _ALPHA * (sum_m - s_negsum)

        if out_copy[0] is not None:
            out_copy[0].wait()
            oidx = plists[(j + 1) % 2]
            onidx = nlists[(j + 1) % 2]
            for t in range(2):
                opi = oidx[pl.ds(t * _L, _L)]
                plsc.store_scatter(out_v, [opi],
                                   jnp.zeros((_L,), jnp.float32))
                oni = onidx[pl.ds(t * _L, _L)]
                plsc.store_scatter(out_v, [oni],
                                   jnp.zeros((_L,), jnp.float32))

        for t in range(2):
            pidx = posidx[pl.ds(t * _L, _L)]
            plsc.store_scatter(out_v, [pidx],
                               pidx.astype(jnp.float32) + pos_tmp)
            nidx = negidx[pl.ds(t * _L, _L)]
            nval = negval[pl.ds(t * _L, _L)]
            plsc.store_scatter(out_v, [nidx], nval - neg_tmp)

        out_copy[0] = pltpu.async_copy(out_v, o_hbm.at[row], osem)

    out_copy[0].wait()


def kernel(x):
    mesh = plsc.VectorSubcoreMesh(core_axis_name="c", subcore_axis_name="s",
                                  num_cores=2, num_subcores=16)
    f = pl.kernel(
        _sc_body,
        out_type=jax.ShapeDtypeStruct((_B, _D), jnp.float32),
        mesh=mesh,
        compiler_params=pltpu.CompilerParams(use_tc_tiling_on_sc=True,
                                             needs_layout_passes=False),
        scratch_types=[
            pltpu.VMEM((_D,), jnp.float32),
            pltpu.VMEM((_D,), jnp.float32),
            pltpu.VMEM((_D,), jnp.float32),
            pltpu.VMEM((_GROUPS * _L,), jnp.float32),
            pltpu.VMEM((_GROUPS * _L,), jnp.float32),
            pltpu.VMEM((32,), jnp.int32),
            pltpu.VMEM((32,), jnp.int32),
            pltpu.VMEM((32,), jnp.int32),
            pltpu.VMEM((32,), jnp.int32),
            pltpu.VMEM((32,), jnp.float32),
            pltpu.VMEM((32,), jnp.float32),
            pltpu.SemaphoreType.DMA,
            pltpu.SemaphoreType.DMA,
            pltpu.SemaphoreType.DMA,
        ],
    )
    return f(x)


# speedup vs baseline: 1.0290x; 1.0290x over previous
import jax
import jax.numpy as jnp
from jax import lax
from jax.experimental import pallas as pl
from jax.experimental.pallas import tpu as pltpu
from jax.experimental.pallas import tpu_sc as plsc

_ALPHA = 6.26
_D = 32768
_B = 128
_L = 16
_GROUPS = 128
_VPG = 16
_NSUB = 32
_ROWS_PER_W = _B // _NSUB
_GV = _GROUPS // _L


def _iota16():
    return lax.broadcasted_iota(jnp.int32, (_L,), 0)


def _splat_i32(x):
    return jnp.broadcast_to(x, (_L,)).astype(jnp.int32)


def _splat_f32(x):
    return jnp.broadcast_to(x, (_L,)).astype(jnp.float32)


def _store1(ref, pos, val, i16):
    plsc.store_scatter(ref, [_splat_i32(pos)], val, mask=i16 == 0)


def _sc_body(x_hbm, o_hbm, row_a, row_b, out_v, maxbuf, minbuf,
             pos_a, pos_b, neg_a, neg_b, nval_a, nval_b,
             sem_a, sem_b, osem):
    i16 = _iota16()
    wid = lax.axis_index("c") * 16 + lax.axis_index("s")
    row0 = wid * _ROWS_PER_W

    pltpu.async_copy(x_hbm.at[row0], row_a, sem_a)

    @plsc.parallel_loop(0, _GROUPS, 1, unroll=2)
    def _zero_block(i):
        for k in range(_VPG):
            out_v[pl.ds(i * (_VPG * _L) + k * _L, _L)] = \
                jnp.zeros((_L,), jnp.float32)

    def process(row, row_v, posidx, negidx, negval,
                zidx, znidx, drain_prev, zero_prev):
        zv = jnp.zeros((_L,), jnp.float32)

        def p1_group(g, carry):
            s_all, s_abs, m_all, mn_all = carry
            base = g * (_VPG * _L)
            v = row_v[pl.ds(base, _L)]
            gmx = v
            gmn = v
            sa = v
            sb = jnp.abs(v)
            for k in range(1, _VPG):
                v = row_v[pl.ds(base + k * _L, _L)]
                gmx = jnp.maximum(gmx, v)
                gmn = jnp.minimum(gmn, v)
                sa = sa + v
                sb = sb + jnp.abs(v)
            plsc.store_scatter(maxbuf, [i16 * _GROUPS + g], gmx)
            plsc.store_scatter(minbuf, [i16 * _GROUPS + g], gmn)
            return (s_all + sa, s_abs + sb,
                    jnp.maximum(m_all, gmx), jnp.minimum(mn_all, gmn))

        s_all, s_abs, m_all, mn_all = plsc.parallel_loop(
            0, _GROUPS, 1, unroll=2,
            carry=(zv, zv, jnp.full((_L,), -jnp.inf, jnp.float32),
                   jnp.full((_L,), jnp.inf, jnp.float32)))(p1_group)
        sum_all = jnp.sum(s_all)
        sum_abs = jnp.sum(s_abs)
        s_pos = 0.5 * (sum_abs + sum_all)
        s_negsum = 0.5 * (sum_all - sum_abs)

        def side(buf, m_vec, idx_list, val_list, is_max, i, acc):
            red = jnp.max if is_max else jnp.min
            m = red(m_vec)
            l_v = plsc.all_reduce_ffs(m_vec == m)
            found = jnp.full((_L,), _GROUPS, jnp.int32)
            gvs = []
            for t in range(_GV):
                gvec = plsc.load_gather(
                    buf, [l_v * _GROUPS + (t * _L) + i16])
                gvs.append(gvec)
                found = jnp.minimum(
                    found, jnp.where(gvec == m, i16 + t * _L, _GROUPS))
            g_v = _splat_i32(jnp.min(found))
            base_v = g_v * (_VPG * _L) + l_v
            didx = base_v + i16 * _L
            v = plsc.load_gather(row_v, [didx])
            kl_v = plsc.all_reduce_ffs(v == m)
            elem_v = base_v + kl_v * _L
            v2 = jnp.where(i16 == kl_v, 0.0, v)
            plsc.store_scatter(row_v, [didx], v2)
            newm = _splat_f32(red(v2))
            _store1(buf, l_v * _GROUPS + g_v, newm, i16)
            _store1(idx_list, i, elem_v, i16)
            if val_list is not None:
                _store1(val_list, i, _splat_f32(m), i16)
            lacc = None
            for t in range(_GV):
                gm = jnp.where(t * _L + i16 == g_v, newm, gvs[t])
                lacc = gm if lacc is None else (
                    jnp.maximum(lacc, gm) if is_max
                    else jnp.minimum(lacc, gm))
            ml = red(lacc)
            m_vec = jnp.where(i16 == l_v, ml, m_vec)
            acc = acc + (elem_v.astype(jnp.float32) if is_max
                         else _splat_f32(m))
            return m_vec, acc

        def step(i, carry):
            mx_vec, accp, mn_vec, accn = carry
            mx_vec, accp = side(maxbuf, mx_vec, posidx, None, True, i, accp)
            mn_vec, accn = side(minbuf, mn_vec, negidx, negval, False, i,
                                accn)
            return (mx_vec, accp, mn_vec, accn)

        _, accp, _, accn = lax.fori_loop(
            0, 32, step, (m_all, zv, mn_all, zv))
        sum_idx = jnp.max(accp)
        sum_m = jnp.min(accn)

        pos_tmp = _ALPHA * (s_pos - sum_idx)
        neg_tmp = _ALPHA * (sum_m - s_negsum)

        if drain_prev:
            def do_drain():
                pltpu.make_async_copy(out_v, o_hbm.at[row], osem).wait()
                for t in range(2):
                    opi = zidx[pl.ds(t * _L, _L)]
                    plsc.store_scatter(out_v, [opi],
                                       jnp.zeros((_L,), jnp.float32))
                    oni = znidx[pl.ds(t * _L, _L)]
                    plsc.store_scatter(out_v, [oni],
                                       jnp.zeros((_L,), jnp.float32))

            if zero_prev is None:
                do_drain()
            else:
                pl.when(zero_prev)(do_drain)

        for t in range(2):
            pidx = posidx[pl.ds(t * _L, _L)]
            plsc.store_scatter(out_v, [pidx],
                               pidx.astype(jnp.float32) + pos_tmp)
            nidx = negidx[pl.ds(t * _L, _L)]
            nval = negval[pl.ds(t * _L, _L)]
            plsc.store_scatter(out_v, [nidx], nval - neg_tmp)

        pltpu.async_copy(out_v, o_hbm.at[row], osem)

    def pair(t, _):
        ra = row0 + 2 * t
        rb = ra + 1

        pltpu.make_async_copy(x_hbm.at[ra], row_a, sem_a).wait()
        pltpu.async_copy(x_hbm.at[rb], row_b, sem_b)
        process(ra, row_a, pos_a, neg_a, nval_a,
                pos_b, neg_b, drain_prev=True, zero_prev=t > 0)

        pltpu.make_async_copy(x_hbm.at[rb], row_b, sem_b).wait()

        @pl.when(t + 1 < _ROWS_PER_W // 2)
        def _():
            pltpu.async_copy(x_hbm.at[rb + 1], row_a, sem_a)

        process(rb, row_b, pos_b, neg_b, nval_b,
                pos_a, neg_a, drain_prev=True, zero_prev=None)
        return 0

    lax.fori_loop(0, _ROWS_PER_W // 2, pair, 0)
    pltpu.make_async_copy(out_v, o_hbm.at[row0], osem).wait()


def kernel(x):
    mesh = plsc.VectorSubcoreMesh(core_axis_name="c", subcore_axis_name="s",
                                  num_cores=2, num_subcores=16)
    f = pl.kernel(
        _sc_body,
        out_type=jax.ShapeDtypeStruct((_B, _D), jnp.float32),
        mesh=mesh,
        compiler_params=pltpu.CompilerParams(use_tc_tiling_on_sc=True,
                                             needs_layout_passes=False),
        scratch_types=[
            pltpu.VMEM((_D,), jnp.float32),
            pltpu.VMEM((_D,), jnp.float32),
            pltpu.VMEM((_D,), jnp.float32),
            pltpu.VMEM((_GROUPS * _L,), jnp.float32),
            pltpu.VMEM((_GROUPS * _L,), jnp.float32),
            pltpu.VMEM((32,), jnp.int32),
            pltpu.VMEM((32,), jnp.int32),
            pltpu.VMEM((32,), jnp.int32),
            pltpu.VMEM((32,), jnp.int32),
            pltpu.VMEM((32,), jnp.float32),
            pltpu.VMEM((32,), jnp.float32),
            pltpu.SemaphoreType.DMA,
            pltpu.SemaphoreType.DMA,
            pltpu.SemaphoreType.DMA,
        ],
    )
    return f(x)
